# baseline (speedup 1.0000x reference)
import functools

import jax
import jax.numpy as jnp
from jax import lax
from jax.experimental import pallas as pl
from jax.experimental.pallas import tpu as pltpu
from jax.experimental.pallas import tpu_sc as plsc

N_NEWS = 50000
N_COMP = 10000
H = 64
E_MEN = 800000
E_CC = 320000

NC = 2
NS = 16
NW = NC * NS
CHUNK = 128

MEN_CHUNKS = 200
CC_CHUNKS = 80
SEG_BLK = 40
NBUF = 8
PREF = 4

ACC_ROWS = 10240
ZSTRIPE = ACC_ROWS // NS
WSTRIPE = 1000
NWT = N_COMP // WSTRIPE

_f32 = jnp.float32


def _mesh():
    return plsc.VectorSubcoreMesh(core_axis_name="c", subcore_axis_name="s",
                                  num_cores=NC, num_subcores=NS)


_SC_PARAMS = pltpu.CompilerParams(use_tc_tiling_on_sc=False)


def _zero_rows(rows_v, n):
    def zr(i, c):
        rows_v[i // 4, pl.ds((i % 4) * 16, 16)] = jnp.zeros((16,), _f32)
        return c
    lax.fori_loop(0, n * 4, zr, 0)


def _fill16(ref, n, value):
    def fb(i, c):
        ref[i, :] = jnp.full((16,), value, _f32)
        return c
    lax.fori_loop(0, n, fb, 0)


def _sc_segsum(table, src2d, dst2d, nbuf, pref, stage_table, c0_frac=None,
               ccdst2d=None):
    n_chunks2 = 2 * (src2d.shape[0] // NW)
    n_table = table.shape[0]
    with_counts = ccdst2d is not None
    cc_chunks = ccdst2d.shape[0] // NW if with_counts else 0
    out_type = [jax.ShapeDtypeStruct((NC, N_COMP, H), _f32)]
    scratch = [
        pltpu.VMEM((SEG_BLK, CHUNK), jnp.int32),
        pltpu.VMEM((SEG_BLK, CHUNK), jnp.int32),
        [pltpu.VMEM((CHUNK, H), _f32)] * nbuf,
        pltpu.VMEM_SHARED((ACC_ROWS, H), _f32),
        [pltpu.SemaphoreType.DMA] * nbuf,
        [pltpu.SemaphoreType.DMA] * nbuf,
    ]
    if stage_table:
        scratch.append(pltpu.VMEM_SHARED((n_table, H), _f32))
    if with_counts:
        out_type += [jax.ShapeDtypeStruct((NC, N_COMP, 16), _f32),
                     jax.ShapeDtypeStruct((NC, N_COMP, 16), _f32)]
        scratch += [
            pltpu.VMEM((CC_CHUNKS, CHUNK), jnp.int32),
            pltpu.VMEM((CHUNK, 16), _f32),
            pltpu.VMEM_SHARED((ACC_ROWS, 16), _f32),
            pltpu.VMEM_SHARED((ACC_ROWS, 16), _f32),
            pltpu.SemaphoreType.DMA,
            pltpu.SemaphoreType.DMA,
        ]

    @functools.partial(
        pl.kernel,
        out_type=out_type if with_counts else out_type[0],
        mesh=_mesh(),
        compiler_params=_SC_PARAMS,
        scratch_types=scratch,
    )
    def k(table_hbm, *args):
        if with_counts:
            (ccdst_hbm, src_hbm, dst_hbm, sum_hbm, cnt_hbm, ccnt_hbm,
             src_v, dst_v, rows, acc_sh, gsem, ssem,
             ccdst_v, ones_v, cnt_sh, ccnt_sh, osem, csem) = args
        elif stage_table:
            (src_hbm, dst_hbm, sum_hbm,
             src_v, dst_v, rows, acc_sh, gsem, ssem, tab_sh) = args
        else:
            (src_hbm, dst_hbm, sum_hbm,
             src_v, dst_v, rows, acc_sh, gsem, ssem) = args
        cid = lax.axis_index("c")
        sid = lax.axis_index("s")
        if c0_frac is None:
            a_chunks = b_chunks = n_chunks2 // 2
        else:
            a_chunks = int(n_chunks2 * c0_frac)
            b_chunks = n_chunks2 - a_chunks
        assert a_chunks % SEG_BLK == 0 and b_chunks % SEG_BLK == 0
        my_base = sid * n_chunks2 + cid * a_chunks
        my_blocks = lax.select(cid == 0, a_chunks // SEG_BLK,
                               b_chunks // SEG_BLK)

        _zero_rows(rows[0], CHUNK)
        base = sid * ZSTRIPE
        def za(k_, c):
            pltpu.sync_copy(rows[0],
                            acc_sh.at[pl.ds(base + k_ * CHUNK, CHUNK), :])
            return c
        lax.fori_loop(0, ZSTRIPE // CHUNK, za, 0)
        if with_counts:
            _fill16(ones_v, CHUNK, 0.0)
            for sh in (cnt_sh, ccnt_sh):
                def zc(k_, c):
                    pltpu.sync_copy(
                        ones_v, sh.at[pl.ds(base + k_ * CHUNK, CHUNK), :])
                    return c
                lax.fori_loop(0, ZSTRIPE // CHUNK, zc, 0)
            _fill16(ones_v, CHUNK, 1.0)
        if stage_table:
            tstripe = n_table // NS
            pltpu.sync_copy(table_hbm.at[pl.ds(sid * tstripe, tstripe), :],
                            tab_sh.at[pl.ds(sid * tstripe, tstripe), :])
        plsc.subcore_barrier()
        gather_src = tab_sh if stage_table else table_hbm

        if with_counts:
            pltpu.sync_copy(
                ccdst_hbm.at[pl.ds((sid * NC + cid) * cc_chunks, cc_chunks),
                             :], ccdst_v)
            def ccb(j, c):
                pltpu.async_copy(ones_v, ccnt_sh.at[ccdst_v.at[j]], csem,
                                 add=True)
                return c
            lax.fori_loop(0, cc_chunks, ccb, 0)

        n_rounds = SEG_BLK // nbuf

        def chunk_step(j, u, drain_scatter, issue_next):
            pltpu.make_async_copy(
                gather_src.at[src_v.at[j]], rows[u], gsem[u]).wait()
            pltpu.async_copy(rows[u], acc_sh.at[dst_v.at[j]], ssem[u],
                             add=True)
            if with_counts:
                pltpu.async_copy(ones_v, cnt_sh.at[dst_v.at[j]], osem,
                                 add=True)
            if issue_next:
                v = (u + pref) % nbuf
                if drain_scatter:
                    pltpu.make_async_copy(rows[v], acc_sh.at[dst_v.at[0]],
                                          ssem[v]).wait()
                pltpu.async_copy(gather_src.at[src_v.at[j + pref]], rows[v],
                                 gsem[v])

        def blk(b, c):
            base2 = my_base + b * SEG_BLK
            pltpu.sync_copy(src_hbm.at[pl.ds(base2, SEG_BLK), :], src_v)
            pltpu.sync_copy(dst_hbm.at[pl.ds(base2, SEG_BLK), :], dst_v)
            for u in range(pref):
                pltpu.async_copy(gather_src.at[src_v.at[u]], rows[u], gsem[u])
            for u in range(nbuf):
                chunk_step(u, u, drain_scatter=(u >= nbuf - pref),
                           issue_next=True)
            def step(jj, c2):
                for u in range(nbuf):
                    chunk_step(jj * nbuf + u, u, drain_scatter=True,
                               issue_next=True)
                return c2
            lax.fori_loop(1, n_rounds - 1, step, 0)
            for u in range(nbuf):
                chunk_step((n_rounds - 1) * nbuf + u, u,
                           drain_scatter=True,
                           issue_next=(u < nbuf - pref))
            for u in range(nbuf):
                pltpu.make_async_copy(rows[u], acc_sh.at[dst_v.at[0]],
                                      ssem[u]).wait()
            if with_counts:
                def dro(i, c2):
                    pltpu.make_async_copy(ones_v, cnt_sh.at[dst_v.at[0]],
                                          osem).wait()
                    return c2
                return lax.fori_loop(0, SEG_BLK, dro, c)
            return c
        lax.fori_loop(0, my_blocks, blk, 0)

        if with_counts:
            def drc(i, c):
                pltpu.make_async_copy(ones_v, ccnt_sh.at[ccdst_v.at[0]],
                                      csem).wait()
                return c
            lax.fori_loop(0, cc_chunks, drc, 0)

        plsc.subcore_barrier()

        @pl.when(sid < NWT)
        def _():
            wbase = sid * WSTRIPE
            pltpu.sync_copy(acc_sh.at[pl.ds(wbase, WSTRIPE), :],
                            sum_hbm.at[cid, pl.ds(wbase, WSTRIPE), :])
            if with_counts:
                pltpu.sync_copy(cnt_sh.at[pl.ds(wbase, WSTRIPE), :],
                                cnt_hbm.at[cid, pl.ds(wbase, WSTRIPE), :])
                pltpu.sync_copy(ccnt_sh.at[pl.ds(wbase, WSTRIPE), :],
                                ccnt_hbm.at[cid, pl.ds(wbase, WSTRIPE), :])

    if with_counts:
        return k(table, ccdst2d, src2d, dst2d)
    return k(table, src2d, dst2d)


def _ln(x, g, b):
    m = jnp.mean(x, axis=-1, keepdims=True)
    xc = x - m
    v = jnp.mean(xc * xc, axis=-1, keepdims=True)
    return xc / jnp.sqrt(v + 1e-5) * g + b


def _tc_news_proj(news_x, Wn, bn):
    BLK = 5000
    def body(x_ref, w_ref, b_ref, o_ref):
        o_ref[...] = jnp.maximum(
            jnp.dot(x_ref[...], w_ref[...], preferred_element_type=_f32)
            + b_ref[...][None, :], 0.0)
    return pl.pallas_call(
        body,
        grid=(N_NEWS // BLK,),
        in_specs=[
            pl.BlockSpec((BLK, 128), lambda i: (i, 0)),
            pl.BlockSpec((128, H), lambda i: (0, 0)),
            pl.BlockSpec((H,), lambda i: (0,)),
        ],
        out_specs=pl.BlockSpec((BLK, H), lambda i: (i, 0)),
        out_shape=jax.ShapeDtypeStruct((N_NEWS, H), _f32),
    )(news_x, Wn, bn)


def _tc_fuse(company_x, Wc, bc, msum2, mcnt2, Wf, bf, gf, betaf):
    def body(cx_ref, wc_ref, bc_ref, ms_ref, mc_ref, wf_ref, bf_ref,
             g_ref, b_ref, o_ref):
        ch = jnp.maximum(
            jnp.dot(cx_ref[...], wc_ref[...], preferred_element_type=_f32)
            + bc_ref[...][None, :], 0.0)
        msum = ms_ref[0] + ms_ref[1]
        cnt = mc_ref[0, :, 0:1] + mc_ref[1, :, 0:1]
        agg = msum / jnp.maximum(cnt, 1.0)
        z = (jnp.dot(ch, wf_ref[0:H, :], preferred_element_type=_f32)
             + jnp.dot(agg, wf_ref[H:2 * H, :], preferred_element_type=_f32)
             + bf_ref[...][None, :])
        o_ref[...] = _ln(jnp.maximum(z, 0.0), g_ref[...][None, :],
                         b_ref[...][None, :])
    return pl.pallas_call(
        body,
        out_shape=jax.ShapeDtypeStruct((N_COMP, H), _f32),
    )(company_x, Wc, bc, msum2, mcnt2, Wf, bf, gf, betaf)


def _tc_conv(s2, cnt2, x, Wl, bl, Wr, g, b):
    def body(s_ref, c_ref, x_ref, wl_ref, bl_ref, wr_ref, g_ref, b_ref, o_ref):
        ssum = s_ref[0] + s_ref[1]
        cnt = c_ref[0, :, 0:1] + c_ref[1, :, 0:1]
        agg = ssum / jnp.maximum(cnt, 1.0)
        z = (jnp.dot(agg, wl_ref[...], preferred_element_type=_f32)
             + bl_ref[...][None, :]
             + jnp.dot(x_ref[...], wr_ref[...], preferred_element_type=_f32))
        o_ref[...] = _ln(jnp.maximum(z, 0.0), g_ref[...][None, :],
                         b_ref[...][None, :])
    return pl.pallas_call(
        body,
        out_shape=jax.ShapeDtypeStruct((N_COMP, H), _f32),
    )(s2, cnt2, x, Wl, bl, Wr, g, b)


def _tc_out(s2, cnt2, x, Wl, bl, Wr, g, b, Wk1, bk1, Wk2, bk2):
    def body(s_ref, c_ref, x_ref, wl_ref, bl_ref, wr_ref, g_ref, b_ref,
             wk1_ref, bk1_ref, wk2_ref, bk2_ref, o_ref):
        ssum = s_ref[0] + s_ref[1]
        cnt = c_ref[0, :, 0:1] + c_ref[1, :, 0:1]
        agg = ssum / jnp.maximum(cnt, 1.0)
        z = (jnp.dot(agg, wl_ref[...], preferred_element_type=_f32)
             + bl_ref[...][None, :]
             + jnp.dot(x_ref[...], wr_ref[...], preferred_element_type=_f32))
        co = _ln(jnp.maximum(z, 0.0), g_ref[...][None, :], b_ref[...][None, :])
        h = jnp.maximum(
            jnp.dot(co, wk1_ref[...], preferred_element_type=_f32)
            + bk1_ref[...][None, :], 0.0)
        o_ref[...] = (jnp.dot(h, wk2_ref[...], preferred_element_type=_f32)
                      + bk2_ref[...][None, :])
    return pl.pallas_call(
        body,
        out_shape=jax.ShapeDtypeStruct((N_COMP, 1), _f32),
    )(s2, cnt2, x, Wl, bl, Wr, g, b, Wk1, bk1, Wk2, bk2)


def _pad_edges(idx, total, pad_value):
    n = total - idx.shape[0]
    return jnp.concatenate(
        [idx.astype(jnp.int32), jnp.full((n,), pad_value, jnp.int32)]
    ).reshape(total // CHUNK, CHUNK)


def kernel(news_x, company_x, edge_attr, mentions_src, mentions_dst,
           nn_edge_index, cc_edge_index,
           Wn, bn, Wc, bc, Wf, bf, gf, betaf,
           W1n_l, b1n, W1n_r, W1c_l, b1c, W1c_r,
           W2n_l, b2n, W2n_r, W2c_l, b2c, W2c_r,
           g_news, beta_news, g_comp, beta_comp,
           Wk1, bk1, Wk2, bk2):
    e_men_pad = NW * MEN_CHUNKS * CHUNK
    e_cc_pad = NW * CC_CHUNKS * CHUNK
    msrc = _pad_edges(mentions_src, e_men_pad, 0)
    mdst = _pad_edges(mentions_dst, e_men_pad, N_COMP)
    ccsrc = _pad_edges(cc_edge_index[0], e_cc_pad, 0)
    ccdst = _pad_edges(cc_edge_index[1], e_cc_pad, N_COMP)

    news_h = _tc_news_proj(news_x, Wn, bn)
    msum2, mcnt2, ccnt2 = _sc_segsum(news_h, msrc, mdst, nbuf=5, pref=3,
                                     stage_table=False, c0_frac=0.9,
                                     ccdst2d=ccdst)
    fused = _tc_fuse(company_x, Wc, bc, msum2, mcnt2, Wf, bf, gf, betaf)
    s1 = _sc_segsum(fused, ccsrc, ccdst, nbuf=4, pref=2, stage_table=True)
    comp1 = _tc_conv(s1, ccnt2, fused, W1c_l, b1c, W1c_r, g_comp, beta_comp)
    s2 = _sc_segsum(comp1, ccsrc, ccdst, nbuf=4, pref=2, stage_table=True)
    logits2 = _tc_out(s2, ccnt2, comp1, W2c_l, b2c, W2c_r, g_comp,
                      beta_comp, Wk1, bk1, Wk2, bk2)
    return logits2[:, 0]


# speedup vs baseline: 1.0945x; 1.0945x over previous
import functools

import jax
import jax.numpy as jnp
from jax import lax
from jax.experimental import pallas as pl
from jax.experimental.pallas import tpu as pltpu
from jax.experimental.pallas import tpu_sc as plsc

N_NEWS = 50000
N_COMP = 10000
H = 64
E_MEN = 800000
E_CC = 320000

NC = 2
NS = 16
NW = NC * NS
CHUNK = 128

MEN_CHUNKS = 200
CC_CHUNKS = 80
SEG_BLK = 40
NBUF = 8
PREF = 4

ACC_ROWS = 10240
ZSTRIPE = ACC_ROWS // NS
WSTRIPE = 1000
NWT = N_COMP // WSTRIPE

_f32 = jnp.float32


def _mesh():
    return plsc.VectorSubcoreMesh(core_axis_name="c", subcore_axis_name="s",
                                  num_cores=NC, num_subcores=NS)


_SC_PARAMS = pltpu.CompilerParams(use_tc_tiling_on_sc=False)


def _zero_rows(rows_v, n):
    def zr(i, c):
        rows_v[i // 4, pl.ds((i % 4) * 16, 16)] = jnp.zeros((16,), _f32)
        return c
    lax.fori_loop(0, n * 4, zr, 0)


def _fill16(ref, n, value):
    def fb(i, c):
        ref[i, :] = jnp.full((16,), value, _f32)
        return c
    lax.fori_loop(0, n, fb, 0)


def _sc_segsum(table, src2d, dst2d, nbuf, pref, stage_table, c0_frac=None,
               ccdst2d=None):
    n_chunks2 = 2 * (src2d.shape[0] // NW)
    n_table = table.shape[0]
    with_counts = ccdst2d is not None
    cc_chunks = ccdst2d.shape[0] // NW if with_counts else 0
    out_type = [jax.ShapeDtypeStruct((NC, N_COMP, H), _f32)]
    scratch = [
        pltpu.VMEM((SEG_BLK, CHUNK), jnp.int32),
        pltpu.VMEM((SEG_BLK, CHUNK), jnp.int32),
        [pltpu.VMEM((CHUNK, H), _f32)] * nbuf,
        pltpu.VMEM_SHARED((ACC_ROWS, H), _f32),
        [pltpu.SemaphoreType.DMA] * nbuf,
        [pltpu.SemaphoreType.DMA] * nbuf,
    ]
    if stage_table:
        scratch.append(pltpu.VMEM_SHARED((n_table, H), _f32))
    if with_counts:
        out_type += [jax.ShapeDtypeStruct((NC, N_COMP, 16), _f32),
                     jax.ShapeDtypeStruct((NC, N_COMP, 16), _f32)]
        scratch += [
            pltpu.VMEM((CC_CHUNKS, CHUNK), jnp.int32),
            pltpu.VMEM((CHUNK, 16), _f32),
            pltpu.VMEM_SHARED((ACC_ROWS, 16), _f32),
            pltpu.VMEM_SHARED((ACC_ROWS, 16), _f32),
            pltpu.SemaphoreType.DMA,
            pltpu.SemaphoreType.DMA,
        ]

    @functools.partial(
        pl.kernel,
        out_type=out_type if with_counts else out_type[0],
        mesh=_mesh(),
        compiler_params=_SC_PARAMS,
        scratch_types=scratch,
    )
    def k(table_hbm, *args):
        if with_counts:
            (ccdst_hbm, src_hbm, dst_hbm, sum_hbm, cnt_hbm, ccnt_hbm,
             src_v, dst_v, rows, acc_sh, gsem, ssem,
             ccdst_v, ones_v, cnt_sh, ccnt_sh, osem, csem) = args
        elif stage_table:
            (src_hbm, dst_hbm, sum_hbm,
             src_v, dst_v, rows, acc_sh, gsem, ssem, tab_sh) = args
        else:
            (src_hbm, dst_hbm, sum_hbm,
             src_v, dst_v, rows, acc_sh, gsem, ssem) = args
        cid = lax.axis_index("c")
        sid = lax.axis_index("s")
        if c0_frac is None:
            a_chunks = b_chunks = n_chunks2 // 2
        else:
            a_chunks = int(n_chunks2 * c0_frac)
            b_chunks = n_chunks2 - a_chunks
        assert a_chunks % SEG_BLK == 0 and b_chunks % SEG_BLK == 0
        my_base = sid * n_chunks2 + cid * a_chunks
        my_blocks = lax.select(cid == 0, a_chunks // SEG_BLK,
                               b_chunks // SEG_BLK)

        _zero_rows(rows[0], CHUNK)
        base = sid * ZSTRIPE
        def za(k_, c):
            pltpu.sync_copy(rows[0],
                            acc_sh.at[pl.ds(base + k_ * CHUNK, CHUNK), :])
            return c
        lax.fori_loop(0, ZSTRIPE // CHUNK, za, 0)
        if with_counts:
            _fill16(ones_v, CHUNK, 0.0)
            for sh in (cnt_sh, ccnt_sh):
                def zc(k_, c):
                    pltpu.sync_copy(
                        ones_v, sh.at[pl.ds(base + k_ * CHUNK, CHUNK), :])
                    return c
                lax.fori_loop(0, ZSTRIPE // CHUNK, zc, 0)
            _fill16(ones_v, CHUNK, 1.0)
        if stage_table:
            tstripe = n_table // NS
            pltpu.sync_copy(table_hbm.at[pl.ds(sid * tstripe, tstripe), :],
                            tab_sh.at[pl.ds(sid * tstripe, tstripe), :])
        plsc.subcore_barrier()
        gather_src = tab_sh if stage_table else table_hbm

        if with_counts:
            pltpu.sync_copy(
                ccdst_hbm.at[pl.ds((sid * NC + cid) * cc_chunks, cc_chunks),
                             :], ccdst_v)
            def ccb(j, c):
                pltpu.async_copy(ones_v, ccnt_sh.at[ccdst_v.at[j]], csem,
                                 add=True)
                return c
            lax.fori_loop(0, cc_chunks, ccb, 0)

        n_rounds = SEG_BLK // nbuf

        def chunk_step(j, u, drain_scatter, issue_next):
            pltpu.make_async_copy(
                gather_src.at[src_v.at[j]], rows[u], gsem[u]).wait()
            pltpu.async_copy(rows[u], acc_sh.at[dst_v.at[j]], ssem[u],
                             add=True)
            if with_counts:
                pltpu.async_copy(ones_v, cnt_sh.at[dst_v.at[j]], osem,
                                 add=True)
            if issue_next:
                v = (u + pref) % nbuf
                if drain_scatter:
                    pltpu.make_async_copy(rows[v], acc_sh.at[dst_v.at[0]],
                                          ssem[v]).wait()
                pltpu.async_copy(gather_src.at[src_v.at[j + pref]], rows[v],
                                 gsem[v])

        def blk(b, c):
            base2 = my_base + b * SEG_BLK
            pltpu.sync_copy(src_hbm.at[pl.ds(base2, SEG_BLK), :], src_v)
            pltpu.sync_copy(dst_hbm.at[pl.ds(base2, SEG_BLK), :], dst_v)
            for u in range(pref):
                pltpu.async_copy(gather_src.at[src_v.at[u]], rows[u], gsem[u])
            for u in range(nbuf):
                chunk_step(u, u, drain_scatter=(u >= nbuf - pref),
                           issue_next=True)
            def step(jj, c2):
                for u in range(nbuf):
                    chunk_step(jj * nbuf + u, u, drain_scatter=True,
                               issue_next=True)
                return c2
            lax.fori_loop(1, n_rounds - 1, step, 0)
            for u in range(nbuf):
                chunk_step((n_rounds - 1) * nbuf + u, u,
                           drain_scatter=True,
                           issue_next=(u < nbuf - pref))
            for u in range(nbuf):
                pltpu.make_async_copy(rows[u], acc_sh.at[dst_v.at[0]],
                                      ssem[u]).wait()
            if with_counts:
                def dro(i, c2):
                    pltpu.make_async_copy(ones_v, cnt_sh.at[dst_v.at[0]],
                                          osem).wait()
                    return c2
                return lax.fori_loop(0, SEG_BLK, dro, c)
            return c
        lax.fori_loop(0, my_blocks, blk, 0)

        if with_counts:
            def drc(i, c):
                pltpu.make_async_copy(ones_v, ccnt_sh.at[ccdst_v.at[0]],
                                      csem).wait()
                return c
            lax.fori_loop(0, cc_chunks, drc, 0)

        plsc.subcore_barrier()

        @pl.when(sid < NWT)
        def _():
            wbase = sid * WSTRIPE
            pltpu.sync_copy(acc_sh.at[pl.ds(wbase, WSTRIPE), :],
                            sum_hbm.at[cid, pl.ds(wbase, WSTRIPE), :])
            if with_counts:
                pltpu.sync_copy(cnt_sh.at[pl.ds(wbase, WSTRIPE), :],
                                cnt_hbm.at[cid, pl.ds(wbase, WSTRIPE), :])
                pltpu.sync_copy(ccnt_sh.at[pl.ds(wbase, WSTRIPE), :],
                                ccnt_hbm.at[cid, pl.ds(wbase, WSTRIPE), :])

    if with_counts:
        return k(table, ccdst2d, src2d, dst2d)
    return k(table, src2d, dst2d)


def _ln(x, g, b):
    m = jnp.mean(x, axis=-1, keepdims=True)
    xc = x - m
    v = jnp.mean(xc * xc, axis=-1, keepdims=True)
    return xc / jnp.sqrt(v + 1e-5) * g + b


def _tc_news_proj(news_x, Wn, bn):
    BLK = 5000
    def body(x_ref, w_ref, b_ref, o_ref):
        o_ref[...] = jnp.maximum(
            jnp.dot(x_ref[...], w_ref[...], preferred_element_type=_f32)
            + b_ref[...][None, :], 0.0)
    return pl.pallas_call(
        body,
        grid=(N_NEWS // BLK,),
        in_specs=[
            pl.BlockSpec((BLK, 128), lambda i: (i, 0)),
            pl.BlockSpec((128, H), lambda i: (0, 0)),
            pl.BlockSpec((H,), lambda i: (0,)),
        ],
        out_specs=pl.BlockSpec((BLK, H), lambda i: (i, 0)),
        out_shape=jax.ShapeDtypeStruct((N_NEWS, H), _f32),
    )(news_x, Wn, bn)


def _tc_fuse(company_x, Wc, bc, msum2, mcnt2, Wf, bf, gf, betaf):
    def body(cx_ref, wc_ref, bc_ref, ms_ref, mc_ref, wf_ref, bf_ref,
             g_ref, b_ref, o_ref):
        ch = jnp.maximum(
            jnp.dot(cx_ref[...], wc_ref[...], preferred_element_type=_f32)
            + bc_ref[...][None, :], 0.0)
        msum = ms_ref[0] + ms_ref[1]
        cnt = mc_ref[0, :, 0:1] + mc_ref[1, :, 0:1]
        agg = msum / jnp.maximum(cnt, 1.0)
        z = (jnp.dot(ch, wf_ref[0:H, :], preferred_element_type=_f32)
             + jnp.dot(agg, wf_ref[H:2 * H, :], preferred_element_type=_f32)
             + bf_ref[...][None, :])
        o_ref[...] = _ln(jnp.maximum(z, 0.0), g_ref[...][None, :],
                         b_ref[...][None, :])
    return pl.pallas_call(
        body,
        out_shape=jax.ShapeDtypeStruct((N_COMP, H), _f32),
    )(company_x, Wc, bc, msum2, mcnt2, Wf, bf, gf, betaf)


def _tc_conv(s2, cnt2, x, Wl, bl, Wr, g, b):
    def body(s_ref, c_ref, x_ref, wl_ref, bl_ref, wr_ref, g_ref, b_ref, o_ref):
        ssum = s_ref[0] + s_ref[1]
        cnt = c_ref[0, :, 0:1] + c_ref[1, :, 0:1]
        agg = ssum / jnp.maximum(cnt, 1.0)
        z = (jnp.dot(agg, wl_ref[...], preferred_element_type=_f32)
             + bl_ref[...][None, :]
             + jnp.dot(x_ref[...], wr_ref[...], preferred_element_type=_f32))
        o_ref[...] = _ln(jnp.maximum(z, 0.0), g_ref[...][None, :],
                         b_ref[...][None, :])
    return pl.pallas_call(
        body,
        out_shape=jax.ShapeDtypeStruct((N_COMP, H), _f32),
    )(s2, cnt2, x, Wl, bl, Wr, g, b)


def _tc_out(s2, cnt2, x, Wl, bl, Wr, g, b, Wk1, bk1, Wk2, bk2):
    def body(s_ref, c_ref, x_ref, wl_ref, bl_ref, wr_ref, g_ref, b_ref,
             wk1_ref, bk1_ref, wk2_ref, bk2_ref, o_ref):
        ssum = s_ref[0] + s_ref[1]
        cnt = c_ref[0, :, 0:1] + c_ref[1, :, 0:1]
        agg = ssum / jnp.maximum(cnt, 1.0)
        z = (jnp.dot(agg, wl_ref[...], preferred_element_type=_f32)
             + bl_ref[...][None, :]
             + jnp.dot(x_ref[...], wr_ref[...], preferred_element_type=_f32))
        co = _ln(jnp.maximum(z, 0.0), g_ref[...][None, :], b_ref[...][None, :])
        h = jnp.maximum(
            jnp.dot(co, wk1_ref[...], preferred_element_type=_f32)
            + bk1_ref[...][None, :], 0.0)
        o_ref[...] = (jnp.dot(h, wk2_ref[...], preferred_element_type=_f32)
                      + bk2_ref[...][None, :])
    return pl.pallas_call(
        body,
        out_shape=jax.ShapeDtypeStruct((N_COMP, 1), _f32),
    )(s2, cnt2, x, Wl, bl, Wr, g, b, Wk1, bk1, Wk2, bk2)


def _pad_edges(idx, total, pad_value):
    n = total - idx.shape[0]
    return jnp.concatenate(
        [idx.astype(jnp.int32), jnp.full((n,), pad_value, jnp.int32)]
    ).reshape(total // CHUNK, CHUNK)


def kernel(news_x, company_x, edge_attr, mentions_src, mentions_dst,
           nn_edge_index, cc_edge_index,
           Wn, bn, Wc, bc, Wf, bf, gf, betaf,
           W1n_l, b1n, W1n_r, W1c_l, b1c, W1c_r,
           W2n_l, b2n, W2n_r, W2c_l, b2c, W2c_r,
           g_news, beta_news, g_comp, beta_comp,
           Wk1, bk1, Wk2, bk2):
    e_men_pad = NW * MEN_CHUNKS * CHUNK
    e_cc_pad = NW * CC_CHUNKS * CHUNK
    msrc = _pad_edges(mentions_src, e_men_pad, 0)
    mdst = _pad_edges(mentions_dst, e_men_pad, N_COMP)
    ccsrc = _pad_edges(cc_edge_index[0], e_cc_pad, 0)
    ccdst = _pad_edges(cc_edge_index[1], e_cc_pad, N_COMP)

    news_h = _tc_news_proj(news_x, Wn, bn)
    msum2, mcnt2, ccnt2 = _sc_segsum(news_h, msrc, mdst, nbuf=5, pref=3,
                                     stage_table=False, c0_frac=0.8,
                                     ccdst2d=ccdst)
    fused = _tc_fuse(company_x, Wc, bc, msum2, mcnt2, Wf, bf, gf, betaf)
    s1 = _sc_segsum(fused, ccsrc, ccdst, nbuf=4, pref=2, stage_table=True)
    comp1 = _tc_conv(s1, ccnt2, fused, W1c_l, b1c, W1c_r, g_comp, beta_comp)
    s2 = _sc_segsum(comp1, ccsrc, ccdst, nbuf=4, pref=2, stage_table=True)
    logits2 = _tc_out(s2, ccnt2, comp1, W2c_l, b2c, W2c_r, g_comp,
                      beta_comp, Wk1, bk1, Wk2, bk2)
    return logits2[:, 0]
